# baseline (speedup 1.0000x reference)
import functools

import jax
import jax.numpy as jnp
from jax import lax
from jax.experimental import pallas as pl
from jax.experimental.pallas import tpu as pltpu
from jax.experimental.pallas import tpu_sc as plsc

H = 768
S = 77
B = 4096
NC, NS = 2, 16
NW = NC * NS
SEQ_PER_W = B // NW
ROWS = B * S
LANES = 16
SP = S

_mesh = plsc.VectorSubcoreMesh(core_axis_name="c", subcore_axis_name="s")


@functools.partial(
    pl.kernel,
    out_type=jax.ShapeDtypeStruct((B, S, H), jnp.float32),
    mesh=_mesh,
    compiler_params=pltpu.CompilerParams(use_tc_tiling_on_sc=False),
    scratch_types=[
        pltpu.VMEM((32, 1, SP), jnp.int32),
        pltpu.VMEM((S, H), jnp.float32),
        pltpu.VMEM((SP, H), jnp.float32),
        pltpu.SemaphoreType.DMA,
    ],
)
def _embed(ids_hbm, tab_hbm, pos_hbm, out_hbm, idx_v, pos_v, buf_v, sem):
    wid = lax.axis_index("s") * NC + lax.axis_index("c")
    seq0 = wid * SEQ_PER_W
    pltpu.sync_copy(pos_hbm, pos_v)

    def per_chunk(ch, carry0):
        base = seq0 + ch * 32
        pltpu.sync_copy(ids_hbm.at[pl.ds(base, 32)], idx_v)

        def per_seq(s, carry):
            pltpu.async_copy(tab_hbm.at[idx_v.at[s, 0]], buf_v, sem).wait()

            def add_row(r, c2):
                for c in range(H // LANES):
                    sl = pl.ds(c * LANES, LANES)
                    buf_v[r, sl] = buf_v[r, sl] + pos_v[r, sl]
                return c2

            lax.fori_loop(0, S, add_row, 0)
            pltpu.sync_copy(buf_v, out_hbm.at[base + s])
            return carry

        return lax.fori_loop(0, 32, per_seq, carry0)

    lax.fori_loop(0, SEQ_PER_W // 32, per_chunk, 0)


def kernel(input_ids, token_embedding, position_embedding):
    ids3 = input_ids.reshape(B, 1, S)
    return _embed(ids3, token_embedding, position_embedding)


# speedup vs baseline: 1.1697x; 1.1697x over previous
import functools

import jax
import jax.numpy as jnp
from jax import lax
from jax.experimental import pallas as pl
from jax.experimental.pallas import tpu as pltpu
from jax.experimental.pallas import tpu_sc as plsc

H = 768
S = 77
B = 4096
NC, NS = 2, 16
SEQ_PER_SUB = B // NS
LANES = 16
NROW = 40
ROW_SPLIT = ((0, NROW), (NROW, S - NROW))

_mesh = plsc.VectorSubcoreMesh(core_axis_name="c", subcore_axis_name="s")


@functools.partial(
    pl.kernel,
    out_type=jax.ShapeDtypeStruct((B, S, H), jnp.float32),
    mesh=_mesh,
    compiler_params=pltpu.CompilerParams(use_tc_tiling_on_sc=False),
    scratch_types=[
        pltpu.VMEM((SEQ_PER_SUB, 1, S), jnp.int32),
        pltpu.VMEM((NROW, H), jnp.float32),
        pltpu.VMEM((NROW, H), jnp.float32),
        pltpu.VMEM((NROW, H), jnp.float32),
        pltpu.SemaphoreType.DMA,
        pltpu.SemaphoreType.DMA,
        pltpu.SemaphoreType.DMA,
        pltpu.SemaphoreType.DMA,
    ],
)
def _embed(ids_hbm, tab_hbm, pos_hbm, out_hbm,
           idx_v, pos_v, buf0, buf1, g0, g1, so0, so1):
    c = lax.axis_index("c")
    sid = lax.axis_index("s")
    seq0 = sid * SEQ_PER_SUB
    pltpu.sync_copy(ids_hbm.at[pl.ds(seq0, SEQ_PER_SUB)], idx_v)

    bufs = (buf0, buf1)
    gsem = (g0, g1)
    ssem = (so0, so1)

    for ci in range(NC):
        r0, nr = ROW_SPLIT[ci]

        @pl.when(c == ci)
        def _():
            pltpu.sync_copy(pos_hbm.at[pl.ds(r0, nr)], pos_v.at[pl.ds(0, nr)])

            def gstart(s_, b):
                pltpu.async_copy(
                    tab_hbm.at[idx_v.at[s_, 0, pl.ds(r0, nr)]],
                    bufs[b].at[pl.ds(0, nr)], gsem[b])

            def gwait(b):
                pltpu.make_async_copy(
                    tab_hbm.at[pl.ds(0, nr)],
                    bufs[b].at[pl.ds(0, nr)], gsem[b]).wait()

            def sstart(s_, b):
                pltpu.async_copy(
                    bufs[b].at[pl.ds(0, nr)],
                    out_hbm.at[seq0 + s_, pl.ds(r0, nr)], ssem[b])

            def swait(s_, b):
                pltpu.make_async_copy(
                    bufs[b].at[pl.ds(0, nr)],
                    out_hbm.at[seq0 + s_, pl.ds(r0, nr)], ssem[b]).wait()

            gstart(0, 0)

            def outer(i2, carry):
                for b in range(2):
                    ob = 1 - b
                    s_ = i2 * 2 + b
                    gwait(b)
                    if b == 0:
                        @pl.when(i2 >= 1)
                        def _():
                            swait(s_ - 1, ob)
                        gstart(s_ + 1, ob)
                    else:
                        swait(s_ - 1, ob)
                        @pl.when(i2 <= (SEQ_PER_SUB // 2) - 2)
                        def _():
                            gstart(s_ + 1, ob)

                    def add_row(r, c2):
                        for g in range(H // LANES):
                            sl = pl.ds(g * LANES, LANES)
                            bufs[b][r, sl] = bufs[b][r, sl] + pos_v[r, sl]
                        return c2

                    lax.fori_loop(0, nr, add_row, 0)
                    sstart(s_, b)
                return carry

            lax.fori_loop(0, SEQ_PER_SUB // 2, outer, 0)
            swait(SEQ_PER_SUB - 1, 1)


def kernel(input_ids, token_embedding, position_embedding):
    ids3 = input_ids.reshape(B, 1, S)
    return _embed(ids3, token_embedding, position_embedding)


# speedup vs baseline: 1.2395x; 1.0596x over previous
import functools

import jax
import jax.numpy as jnp
from jax import lax
from jax.experimental import pallas as pl
from jax.experimental.pallas import tpu as pltpu
from jax.experimental.pallas import tpu_sc as plsc

H = 768
S = 77
SP = 80
B = 4096
NC, NS = 2, 16
SEQ_PER_SUB = B // NS
LANES = 16
NR = SP // NC
IDX_PER_W = SEQ_PER_SUB * NR

_mesh = plsc.VectorSubcoreMesh(core_axis_name="c", subcore_axis_name="s")


@functools.partial(
    pl.kernel,
    out_type=jax.ShapeDtypeStruct((B, SP, H), jnp.float32),
    mesh=_mesh,
    scratch_types=[
        pltpu.VMEM((IDX_PER_W,), jnp.int32),
        pltpu.VMEM((NR, H), jnp.float32),
        pltpu.VMEM((NR, H), jnp.float32),
        pltpu.VMEM((NR, H), jnp.float32),
        pltpu.SemaphoreType.DMA,
        pltpu.SemaphoreType.DMA,
        pltpu.SemaphoreType.DMA,
        pltpu.SemaphoreType.DMA,
    ],
)
def _embed(idsw_hbm, tab_hbm, pos_hbm, out_hbm,
           idx_v, pos_v, buf0, buf1, g0, g1, so0, so1):
    c = lax.axis_index("c")
    sid = lax.axis_index("s")
    seq0 = sid * SEQ_PER_SUB
    woff = pl.multiple_of((sid * NC + c) * IDX_PER_W, 8)
    pltpu.sync_copy(idsw_hbm.at[pl.ds(woff, IDX_PER_W)], idx_v)

    bufs = (buf0, buf1)
    gsem = (g0, g1)
    ssem = (so0, so1)

    for ci in range(NC):
        r0 = ci * NR

        @pl.when(c == ci)
        def _():
            pltpu.sync_copy(pos_hbm.at[pl.ds(r0, NR)], pos_v)

            def gstart(i, b):
                off = pl.multiple_of(i * NR, 8)
                pltpu.async_copy(
                    tab_hbm.at[idx_v.at[pl.ds(off, NR)]], bufs[b], gsem[b])

            def gwait(b):
                pltpu.make_async_copy(
                    tab_hbm.at[pl.ds(0, NR)], bufs[b], gsem[b]).wait()

            def sstart(i, b):
                pltpu.async_copy(
                    bufs[b], out_hbm.at[seq0 + i, pl.ds(r0, NR)], ssem[b])

            def swait(i, b):
                pltpu.make_async_copy(
                    bufs[b], out_hbm.at[seq0 + i, pl.ds(r0, NR)],
                    ssem[b]).wait()

            gstart(0, 0)

            def outer(i2, carry):
                for b in range(2):
                    ob = 1 - b
                    i = i2 * 2 + b
                    gwait(b)
                    if b == 0:
                        @pl.when(i2 >= 1)
                        def _():
                            swait(i - 1, ob)
                        gstart(i + 1, ob)
                    else:
                        swait(i - 1, ob)
                        @pl.when(i2 <= (SEQ_PER_SUB // 2) - 2)
                        def _():
                            gstart(i + 1, ob)

                    def add_row(r, c2):
                        for g in range(H // LANES):
                            sl = pl.ds(g * LANES, LANES)
                            bufs[b][r, sl] = bufs[b][r, sl] + pos_v[r, sl]
                        return c2

                    lax.fori_loop(0, NR, add_row, 0)
                    sstart(i, b)
                return carry

            lax.fori_loop(0, SEQ_PER_SUB // 2, outer, 0)
            swait(SEQ_PER_SUB - 1, 1)


def kernel(input_ids, token_embedding, position_embedding):
    ids_pad = jnp.pad(input_ids, ((0, 0), (0, SP - S)))
    ids_w = (ids_pad.reshape(NS, SEQ_PER_SUB, NC, NR)
             .transpose(0, 2, 1, 3).reshape(-1))
    pos_pad = jnp.pad(position_embedding, ((0, SP - S), (0, 0)))
    out = _embed(ids_w, token_embedding, pos_pad)
    return out[:, :S, :]
